# baseline (speedup 1.0000x reference)
import jax
import jax.numpy as jnp
from jax.experimental import pallas as pl
from jax.experimental.pallas import tpu as pltpu

_ROWS_PER_BLOCK = 2048


def _add_kernel(x_ref, e_ref, o_ref):
    o_ref[...] = x_ref[...] + e_ref[...]


def kernel(inputs, embeddings):
    B, S, D = inputs.shape
    bs = _ROWS_PER_BLOCK
    sblk = S // bs
    x = inputs.reshape(B * S, D)
    out = pl.pallas_call(
        _add_kernel,
        grid=(sblk, B),
        in_specs=[
            pl.BlockSpec((bs, D), lambda s, b: (b * sblk + s, 0)),
            pl.BlockSpec((bs, D), lambda s, b: (s, 0)),
        ],
        out_specs=pl.BlockSpec((bs, D), lambda s, b: (b * sblk + s, 0)),
        out_shape=jax.ShapeDtypeStruct((B * S, D), inputs.dtype),
        compiler_params=pltpu.CompilerParams(
            dimension_semantics=("parallel", "arbitrary")
        ),
    )(x, embeddings)
    return out.reshape(B, S, D)
